# speedup vs baseline: 4.5113x; 4.5113x over previous
import functools

import jax
import jax.numpy as jnp
from jax import lax
from jax.experimental import pallas as pl
from jax.experimental.pallas import tpu as pltpu
from jax.experimental.pallas import tpu_sc as plsc

N = 10000
D = 128
NC = 2
NS = 16
NW = NC * NS
N_PAD = 10240
ROWS_PER_SUB = N_PAD // NS
CH = 128
RB = 1280


def _mesh():
    return plsc.VectorSubcoreMesh(core_axis_name="c", subcore_axis_name="s")


def _deg_body(src_hbm, dst_hbm, zc_hbm, out_hbm,
              src_v, dst_v, ones_v, outc_sh, inc_sh):
    c = lax.axis_index("c")
    s = lax.axis_index("s")
    wid = s * NC + c
    nch = src_v.shape[0]
    pltpu.sync_copy(src_hbm.at[wid], src_v)
    pltpu.sync_copy(dst_hbm.at[wid], dst_v)
    for i in range(CH // 16):
        ones_v[pl.ds(16 * i, 16)] = jnp.ones((16,), jnp.float32)
    t0 = s * ROWS_PER_SUB
    pltpu.sync_copy(zc_hbm.at[0, pl.ds(t0, ROWS_PER_SUB)],
                    outc_sh.at[pl.ds(t0, ROWS_PER_SUB)])
    pltpu.sync_copy(zc_hbm.at[1, pl.ds(t0, ROWS_PER_SUB)],
                    inc_sh.at[pl.ds(t0, ROWS_PER_SUB)])
    plsc.subcore_barrier()

    @pl.loop(0, nch)
    def _(j):
        pltpu.sync_copy(ones_v, outc_sh.at[src_v.at[j]], add=True)
        pltpu.sync_copy(ones_v, inc_sh.at[dst_v.at[j]], add=True)

    plsc.subcore_barrier()
    pltpu.sync_copy(outc_sh.at[pl.ds(t0, ROWS_PER_SUB)],
                    out_hbm.at[c, 0, pl.ds(t0, ROWS_PER_SUB)])
    pltpu.sync_copy(inc_sh.at[pl.ds(t0, ROWS_PER_SUB)],
                    out_hbm.at[c, 1, pl.ds(t0, ROWS_PER_SUB)])


def _agg_body(h_hbm, src_hbm, dst_hbm, zagg_hbm, out_hbm,
              src_v, dst_v, rows_v, agg_sh):
    c = lax.axis_index("c")
    s = lax.axis_index("s")
    wid = s * NC + c
    nch = src_v.shape[0]
    pltpu.sync_copy(src_hbm.at[wid], src_v)
    pltpu.sync_copy(dst_hbm.at[wid], dst_v)
    t0 = s * ROWS_PER_SUB
    pltpu.sync_copy(zagg_hbm.at[pl.ds(t0, ROWS_PER_SUB)],
                    agg_sh.at[pl.ds(t0, ROWS_PER_SUB)])
    plsc.subcore_barrier()

    @pl.loop(0, nch)
    def _(j):
        pltpu.sync_copy(h_hbm.at[src_v.at[j]], rows_v)
        pltpu.sync_copy(rows_v, agg_sh.at[dst_v.at[j]], add=True)

    plsc.subcore_barrier()
    pltpu.sync_copy(agg_sh.at[pl.ds(t0, ROWS_PER_SUB)],
                    out_hbm.at[c, pl.ds(t0, ROWS_PER_SUB)])


def _scale_body(x_ref, dp_ref, o_ref):
    d = dp_ref[0] + dp_ref[1]
    o_ref[...] = x_ref[...] * lax.rsqrt(jnp.maximum(d, 1.0))


def _final_body(p_ref, w_ref, dp_ref, b_ref, o_ref):
    agg = p_ref[0] + p_ref[1]
    rst = jnp.dot(agg, w_ref[...], preferred_element_type=jnp.float32)
    d = dp_ref[0] + dp_ref[1]
    o_ref[...] = rst * lax.rsqrt(jnp.maximum(d, 1.0)) + b_ref[...]


def kernel(feat, edge_index, weight, bias):
    e = edge_index.shape[1]
    nch = -(-e // (NW * CH))
    e_pad = NW * nch * CH
    pad = jnp.full((e_pad - e,), N, jnp.int32)
    src3 = jnp.concatenate([edge_index[0], pad]).reshape(NW, nch, CH)
    dst3 = jnp.concatenate([edge_index[1], pad]).reshape(NW, nch, CH)
    feat_pad = jnp.pad(feat, ((0, N_PAD - N), (0, 0)))
    zc = jnp.zeros((2, N_PAD), jnp.float32)
    zagg = jnp.zeros((N_PAD, D), jnp.float32)

    deg_fn = functools.partial(
        pl.kernel,
        out_type=jax.ShapeDtypeStruct((NC, 2, N_PAD), jnp.float32),
        mesh=_mesh(),
        scratch_types=[
            pltpu.VMEM((nch, CH), jnp.int32),
            pltpu.VMEM((nch, CH), jnp.int32),
            pltpu.VMEM((CH,), jnp.float32),
            pltpu.VMEM_SHARED((N_PAD,), jnp.float32),
            pltpu.VMEM_SHARED((N_PAD,), jnp.float32),
        ],
    )(_deg_body)
    degp = deg_fn(src3, dst3, zc)
    odeg = degp[:, 0, :][..., None]
    ideg = degp[:, 1, :][..., None]

    h = pl.pallas_call(
        _scale_body,
        grid=(N_PAD // RB,),
        in_specs=[
            pl.BlockSpec((RB, D), lambda b: (b, 0)),
            pl.BlockSpec((NC, RB, 1), lambda b: (0, b, 0)),
        ],
        out_specs=pl.BlockSpec((RB, D), lambda b: (b, 0)),
        out_shape=jax.ShapeDtypeStruct((N_PAD, D), jnp.float32),
    )(feat_pad, odeg)

    agg_fn = functools.partial(
        pl.kernel,
        out_type=jax.ShapeDtypeStruct((NC, N_PAD, D), jnp.float32),
        mesh=_mesh(),
        scratch_types=[
            pltpu.VMEM((nch, CH), jnp.int32),
            pltpu.VMEM((nch, CH), jnp.int32),
            pltpu.VMEM((CH, D), jnp.float32),
            pltpu.VMEM_SHARED((N_PAD, D), jnp.float32),
        ],
    )(_agg_body)
    aggp = agg_fn(h, src3, dst3, zagg)

    out = pl.pallas_call(
        _final_body,
        grid=(N_PAD // RB,),
        in_specs=[
            pl.BlockSpec((NC, RB, D), lambda b: (0, b, 0)),
            pl.BlockSpec((D, D), lambda b: (0, 0)),
            pl.BlockSpec((NC, RB, 1), lambda b: (0, b, 0)),
            pl.BlockSpec((1, D), lambda b: (0, 0)),
        ],
        out_specs=pl.BlockSpec((RB, D), lambda b: (b, 0)),
        out_shape=jax.ShapeDtypeStruct((N_PAD, D), jnp.float32),
    )(aggp, weight, ideg, bias[None, :])
    return out[:N]
